# baseline (speedup 1.0000x reference)
import functools

import jax
import jax.numpy as jnp
from jax import lax
from jax.experimental import pallas as pl
from jax.experimental.pallas import tpu as pltpu
from jax.experimental.pallas import tpu_sc as plsc

N = 10000
D = 128
E = 320000
NEG_SLOPE = 0.2

NC = 2
NS = 16
NW = NC * NS
EPW = E // NW
CH = 80
NCH = EPW // CH
RPT = N // NS
ROWS_BLK = 1000


def _proj_body(feat_ref, w_ref, al_ref, ar_ref, fs_ref, el_ref, er_ref):
    x = feat_ref[...]
    w = w_ref[...]
    fs = lax.dot_general(x, w, (((1,), (1,)), ((), ())),
                         preferred_element_type=jnp.float32)
    fs_ref[...] = fs
    el = jnp.sum(fs * al_ref[...], axis=1, keepdims=True)
    er = jnp.sum(fs * ar_ref[...], axis=1, keepdims=True)
    el_ref[...] = jnp.broadcast_to(el, el_ref.shape)
    er_ref[...] = jnp.broadcast_to(er, er_ref.shape)


def _project(feat, w_fc, attn_l, attn_r):
    grid = (N // ROWS_BLK,)
    return pl.pallas_call(
        _proj_body,
        grid=grid,
        in_specs=[
            pl.BlockSpec((ROWS_BLK, D), lambda i: (i, 0)),
            pl.BlockSpec((D, D), lambda i: (0, 0)),
            pl.BlockSpec((1, D), lambda i: (0, 0)),
            pl.BlockSpec((1, D), lambda i: (0, 0)),
        ],
        out_specs=[
            pl.BlockSpec((ROWS_BLK, D), lambda i: (i, 0)),
            pl.BlockSpec((ROWS_BLK, 8), lambda i: (i, 0)),
            pl.BlockSpec((ROWS_BLK, 8), lambda i: (i, 0)),
        ],
        out_shape=[
            jax.ShapeDtypeStruct((N, D), jnp.float32),
            jax.ShapeDtypeStruct((N, 8), jnp.float32),
            jax.ShapeDtypeStruct((N, 8), jnp.float32),
        ],
    )(feat, w_fc, attn_l, attn_r)


def _edge_kernel_body(fs_hbm, el_hbm, er_hbm, eidx_hbm,
                      acc_out, den_out,
                      el_v, er_v, den_v, idx2, buf_a, buf_b,
                      gsem_a, gsem_b, ssem_a, ssem_b, acc_sh):
    cid = lax.axis_index("c")
    sid = lax.axis_index("s")
    wid = cid * NS + sid
    bufs = (buf_a, buf_b)
    gsems = (gsem_a, gsem_b)
    ssems = (ssem_a, ssem_b)

    pltpu.sync_copy(el_hbm, el_v)
    pltpu.sync_copy(er_hbm, er_v)

    zero16 = jnp.zeros((16,), jnp.float32)

    def _zero_buf(r, c):
        for j in range(D // 16):
            buf_a[r, pl.ds(j * 16, 16)] = zero16
        return c
    lax.fori_loop(0, CH, _zero_buf, 0)

    def _zero_den(i, c):
        den_v[pl.ds(i * 16, 16)] = zero16
        return c
    lax.fori_loop(0, N // 16, _zero_den, 0)

    for k in range((N // CH + NS - 1) // NS):
        c = sid + k * NS
        @pl.when(c < N // CH)
        def _z():
            off = pl.multiple_of(c * CH, 8)
            pltpu.sync_copy(buf_a, acc_sh.at[pl.ds(off, CH)])

    plsc.subcore_barrier()

    def _gather_start(k, p):
        pltpu.sync_copy(eidx_hbm.at[wid, k], idx2.at[p])
        pltpu.async_copy(fs_hbm.at[idx2.at[p, 0]], bufs[p], gsems[p])

    def _gather_wait(p):
        pltpu.make_async_copy(fs_hbm.at[idx2.at[p, 0]], bufs[p],
                              gsems[p]).wait()

    def _scatter_start(p):
        pltpu.async_copy(bufs[p], acc_sh.at[idx2.at[p, 1]], ssems[p],
                         add=True)

    def _scatter_wait(p):
        pltpu.make_async_copy(bufs[p], acc_sh.at[idx2.at[p, 1]],
                              ssems[p]).wait()

    def _scale(p):
        buf = bufs[p]

        def _q(q, cc):
            sv = idx2[p, 0, pl.ds(q * 16, 16)]
            dv = idx2[p, 1, pl.ds(q * 16, 16)]
            e = plsc.load_gather(el_v, [sv]) + plsc.load_gather(er_v, [dv])
            e = jnp.where(e > 0, e, NEG_SLOPE * e)
            wv = jnp.exp(e)
            plsc.addupdate_scatter(den_v, [dv], wv)
            for i in range(16):
                r = q * 16 + i
                ws = wv[i]
                for j in range(D // 16):
                    buf[r, pl.ds(j * 16, 16)] = buf[r, pl.ds(j * 16, 16)] * ws
            return cc
        lax.fori_loop(0, CH // 16, _q, 0)

    def _pipe_iter(k, p, first):
        if not first:
            _scatter_wait(p)
        _gather_start(k, p)
        _gather_wait(1 - p)
        _scale(1 - p)
        _scatter_start(1 - p)

    _gather_start(0, 0)
    _pipe_iter(1, 1, True)

    def _pair(kk, c):
        k = 2 + 2 * kk
        _pipe_iter(k, 0, False)
        _pipe_iter(k + 1, 1, False)
        return c
    lax.fori_loop(0, (NCH - 3) // 2, _pair, 0)

    _pipe_iter(NCH - 1, (NCH - 1) & 1, False)
    last = (NCH - 1) & 1
    _gather_wait(last)
    _scale(last)
    _scatter_start(last)
    _scatter_wait(1 - last)
    _scatter_wait(last)

    plsc.subcore_barrier()

    for k in range((N // CH + NS - 1) // NS):
        c = sid + k * NS
        @pl.when(c < N // CH)
        def _wb():
            off = pl.multiple_of(c * CH, 8)
            pltpu.sync_copy(acc_sh.at[pl.ds(off, CH)],
                            acc_out.at[cid, pl.ds(off, CH)])
    pltpu.sync_copy(den_v, den_out.at[wid])


_edge_kernel = functools.partial(
    pl.kernel,
    out_type=(jax.ShapeDtypeStruct((NC, N, D), jnp.float32),
              jax.ShapeDtypeStruct((NW, N), jnp.float32)),
    mesh=plsc.VectorSubcoreMesh(core_axis_name="c", subcore_axis_name="s"),
    compiler_params=pltpu.CompilerParams(needs_layout_passes=False,
                                         use_tc_tiling_on_sc=False),
    scratch_types=[
        pltpu.VMEM((N,), jnp.float32),
        pltpu.VMEM((N,), jnp.float32),
        pltpu.VMEM((N,), jnp.float32),
        pltpu.VMEM((2, 2, CH), jnp.int32),
        pltpu.VMEM((CH, D), jnp.float32),
        pltpu.VMEM((CH, D), jnp.float32),
        pltpu.SemaphoreType.DMA,
        pltpu.SemaphoreType.DMA,
        pltpu.SemaphoreType.DMA,
        pltpu.SemaphoreType.DMA,
        pltpu.VMEM_SHARED((N, D), jnp.float32),
    ],
)(_edge_kernel_body)


def _norm_body(acc_ref, den_ref, out_ref):
    a = acc_ref[0] + acc_ref[1]
    d = jnp.sum(den_ref[...], axis=1)
    inv = jnp.where(d > 0, 1.0 / d, 0.0)
    out_ref[...] = a * inv[:, None]


def _normalize(acc, den):
    grid = (N // ROWS_BLK,)
    return pl.pallas_call(
        _norm_body,
        grid=grid,
        in_specs=[
            pl.BlockSpec((NC, ROWS_BLK, D), lambda i: (0, i, 0)),
            pl.BlockSpec((ROWS_BLK, NW), lambda i: (i, 0)),
        ],
        out_specs=pl.BlockSpec((ROWS_BLK, D), lambda i: (i, 0)),
        out_shape=jax.ShapeDtypeStruct((N, D), jnp.float32),
    )(acc, den)


def kernel(feat, edge_index, W_fc, attn_l, attn_r):
    fs, el8, er8 = _project(feat, W_fc, attn_l, attn_r)
    el = el8[:, 0]
    er = er8[:, 0]
    eidx = jnp.stack(
        [edge_index[0].reshape(NW, NCH, CH),
         edge_index[1].reshape(NW, NCH, CH)], axis=2)
    acc, den = _edge_kernel(fs, el, er, eidx)
    return _normalize(acc, den.T)


# speedup vs baseline: 1.1938x; 1.1938x over previous
import functools

import jax
import jax.numpy as jnp
from jax import lax
from jax.experimental import pallas as pl
from jax.experimental.pallas import tpu as pltpu
from jax.experimental.pallas import tpu_sc as plsc

N = 10000
D = 128
E = 320000
NEG_SLOPE = 0.2

NC = 2
NS = 16
NW = NC * NS
EPW = E // NW
CH = 80
NCH = EPW // CH
RPT = N // NS
ROWS_BLK = 1000


def _proj_body(feat_ref, w_ref, al_ref, ar_ref, fs_ref, el_ref, er_ref):
    x = feat_ref[...]
    w = w_ref[...]
    fs = lax.dot_general(x, w, (((1,), (1,)), ((), ())),
                         preferred_element_type=jnp.float32)
    fs_ref[...] = fs
    el = jnp.sum(fs * al_ref[...], axis=1, keepdims=True)
    er = jnp.sum(fs * ar_ref[...], axis=1, keepdims=True)
    el_ref[...] = jnp.broadcast_to(el, el_ref.shape)
    er_ref[...] = jnp.broadcast_to(er, er_ref.shape)


def _project(feat, w_fc, attn_l, attn_r):
    grid = (N // ROWS_BLK,)
    return pl.pallas_call(
        _proj_body,
        grid=grid,
        in_specs=[
            pl.BlockSpec((ROWS_BLK, D), lambda i: (i, 0)),
            pl.BlockSpec((D, D), lambda i: (0, 0)),
            pl.BlockSpec((1, D), lambda i: (0, 0)),
            pl.BlockSpec((1, D), lambda i: (0, 0)),
        ],
        out_specs=[
            pl.BlockSpec((ROWS_BLK, D), lambda i: (i, 0)),
            pl.BlockSpec((ROWS_BLK, 8), lambda i: (i, 0)),
            pl.BlockSpec((ROWS_BLK, 8), lambda i: (i, 0)),
        ],
        out_shape=[
            jax.ShapeDtypeStruct((N, D), jnp.float32),
            jax.ShapeDtypeStruct((N, 8), jnp.float32),
            jax.ShapeDtypeStruct((N, 8), jnp.float32),
        ],
    )(feat, w_fc, attn_l, attn_r)


def _edge_kernel_body(fs_hbm, elr_hbm, eidx_hbm,
                      acc_out, den_out,
                      elr_v, idxo, w_c, buf_a, buf_b,
                      gsem_a, gsem_b, ssem_a, ssem_b,
                      isem0, isem1, isem2, isem3, dsem_a, dsem_b,
                      acc_sh, den_sh):
    cid = lax.axis_index("c")
    sid = lax.axis_index("s")
    wid = cid * NS + sid
    bufs = (buf_a, buf_b)
    gsems = (gsem_a, gsem_b)
    ssems = (ssem_a, ssem_b)
    isems = (isem0, isem1, isem2, isem3)
    dsems = (dsem_a, dsem_b)

    pltpu.sync_copy(elr_hbm, elr_v)

    zero16 = jnp.zeros((16,), jnp.float32)
    czero = jnp.zeros((16,), jnp.int32)
    cone = czero + 1

    for q in range(2 * CH // 16):
        w_c[q // (CH // 16), pl.ds((q % (CH // 16)) * 16, 16)] = zero16

    def _zero_buf(r, c):
        for j in range(D // 16):
            buf_a[r, pl.ds(j * 16, 16)] = zero16
        return c
    lax.fori_loop(0, CH, _zero_buf, 0)

    NZ = N // CH
    for k in range((NZ + NS - 1) // NS):
        c = sid + k * NS
        @pl.when(c < NZ)
        def _z():
            off = pl.multiple_of(c * CH, 8)
            pltpu.async_copy(buf_a, acc_sh.at[pl.ds(off, CH)], gsem_a)
            pltpu.async_copy(w_c.at[0], den_sh.at[pl.ds(off, CH)], gsem_b)
    for k in range((NZ + NS - 1) // NS):
        c = sid + k * NS
        @pl.when(c < NZ)
        def _zw():
            off = pl.multiple_of(c * CH, 8)
            pltpu.make_async_copy(buf_a, acc_sh.at[pl.ds(off, CH)],
                                  gsem_a).wait()
            pltpu.make_async_copy(w_c.at[0], den_sh.at[pl.ds(off, CH)],
                                  gsem_b).wait()

    plsc.subcore_barrier()

    def _idx_sync(k, s):
        pltpu.sync_copy(eidx_hbm.at[wid, k], idxo.at[s])

    def _idx_start(k, s):
        pltpu.async_copy(eidx_hbm.at[wid, k], idxo.at[s], isems[s])

    def _idx_wait(k, s):
        pltpu.make_async_copy(eidx_hbm.at[wid, k], idxo.at[s],
                              isems[s]).wait()

    def _gather_start(s, p):
        pltpu.async_copy(fs_hbm.at[idxo.at[s, 0]], bufs[p], gsems[p])

    def _gather_wait(s, p):
        pltpu.make_async_copy(fs_hbm.at[idxo.at[s, 0]], bufs[p],
                              gsems[p]).wait()

    def _scatter_start(s, p):
        pltpu.async_copy(bufs[p], acc_sh.at[idxo.at[s, 1]], ssems[p],
                         add=True)

    def _scatter_wait(s, p):
        pltpu.make_async_copy(bufs[p], acc_sh.at[idxo.at[s, 1]],
                              ssems[p]).wait()

    def _den_start(s, p):
        pltpu.async_copy(w_c.at[p], den_sh.at[idxo.at[s, 1]], dsems[p],
                         add=True)

    def _den_wait(s, p):
        pltpu.make_async_copy(w_c.at[p], den_sh.at[idxo.at[s, 1]],
                              dsems[p]).wait()

    def _scale(s, p):
        buf = bufs[p]

        def _q(q, cc):
            sv = idxo[s, 0, pl.ds(q * 16, 16)]
            dv = idxo[s, 1, pl.ds(q * 16, 16)]
            e = (plsc.load_gather(elr_v, [czero, sv])
                 + plsc.load_gather(elr_v, [cone, dv]))
            e = jnp.where(e > 0, e, NEG_SLOPE * e)
            wv = jnp.exp(e)
            w_c[p, pl.ds(q * 16, 16)] = wv
            for i in range(16):
                r = q * 16 + i
                ws = wv[i]
                for j in range(D // 16):
                    buf[r, pl.ds(j * 16, 16)] = buf[r, pl.ds(j * 16, 16)] * ws
            return cc
        lax.fori_loop(0, CH // 16, _q, 0)

    def _pipe(k, s, p, prefetch=True):
        sprev = (s + 3) & 3
        sprev2 = (s + 2) & 3
        _scatter_wait(sprev2, p)
        _den_wait(sprev2, p)
        _idx_wait(k, s)
        _gather_start(s, p)
        if prefetch:
            _idx_start(k + 2, sprev2)
        _gather_wait(sprev, 1 - p)
        _scale(sprev, 1 - p)
        _scatter_start(sprev, 1 - p)
        _den_start(sprev, 1 - p)

    _idx_sync(0, 0)
    _idx_sync(1, 1)
    _gather_start(0, 0)
    _idx_start(2, 2)
    _idx_start(3, 3)
    _gather_start(1, 1)
    _gather_wait(0, 0)
    _scale(0, 0)
    _scatter_start(0, 0)
    _den_start(0, 0)

    def _quad(t, c):
        k0 = 2 + 4 * t
        _pipe(k0, 2, 0)
        _pipe(k0 + 1, 3, 1)
        _pipe(k0 + 2, 0, 0)
        _pipe(k0 + 3, 1, 1)
        return c
    lax.fori_loop(0, (NCH - 5) // 4, _quad, 0)

    _pipe(NCH - 3, (NCH - 3) & 3, (NCH - 3) & 1)
    _pipe(NCH - 2, (NCH - 2) & 3, (NCH - 2) & 1, prefetch=False)
    _pipe(NCH - 1, (NCH - 1) & 3, (NCH - 1) & 1, prefetch=False)
    sl, pl_ = (NCH - 1) & 3, (NCH - 1) & 1
    _gather_wait(sl, pl_)
    _scale(sl, pl_)
    _scatter_start(sl, pl_)
    _den_start(sl, pl_)
    _scatter_wait((NCH - 2) & 3, (NCH - 2) & 1)
    _den_wait((NCH - 2) & 3, (NCH - 2) & 1)
    _scatter_wait(sl, pl_)
    _den_wait(sl, pl_)

    plsc.subcore_barrier()

    for k in range((NZ + NS - 1) // NS):
        c = sid + k * NS
        @pl.when(c < NZ)
        def _wb():
            off = pl.multiple_of(c * CH, 8)
            pltpu.async_copy(acc_sh.at[pl.ds(off, CH)],
                             acc_out.at[cid, pl.ds(off, CH)], gsem_a)
            pltpu.async_copy(den_sh.at[pl.ds(off, CH)],
                             den_out.at[cid, pl.ds(off, CH)], gsem_b)
    for k in range((NZ + NS - 1) // NS):
        c = sid + k * NS
        @pl.when(c < NZ)
        def _wbw():
            off = pl.multiple_of(c * CH, 8)
            pltpu.make_async_copy(acc_sh.at[pl.ds(off, CH)],
                                  acc_out.at[cid, pl.ds(off, CH)],
                                  gsem_a).wait()
            pltpu.make_async_copy(den_sh.at[pl.ds(off, CH)],
                                  den_out.at[cid, pl.ds(off, CH)],
                                  gsem_b).wait()


_edge_kernel = functools.partial(
    pl.kernel,
    out_type=(jax.ShapeDtypeStruct((NC, N, D), jnp.float32),
              jax.ShapeDtypeStruct((NC, N), jnp.float32)),
    mesh=plsc.VectorSubcoreMesh(core_axis_name="c", subcore_axis_name="s"),
    compiler_params=pltpu.CompilerParams(needs_layout_passes=False,
                                         use_tc_tiling_on_sc=False),
    scratch_types=[
        pltpu.VMEM((2, N), jnp.float32),
        pltpu.VMEM((4, 2, CH), jnp.int32),
        pltpu.VMEM((2, CH), jnp.float32),
        pltpu.VMEM((CH, D), jnp.float32),
        pltpu.VMEM((CH, D), jnp.float32),
        pltpu.SemaphoreType.DMA,
        pltpu.SemaphoreType.DMA,
        pltpu.SemaphoreType.DMA,
        pltpu.SemaphoreType.DMA,
        pltpu.SemaphoreType.DMA,
        pltpu.SemaphoreType.DMA,
        pltpu.SemaphoreType.DMA,
        pltpu.SemaphoreType.DMA,
        pltpu.SemaphoreType.DMA,
        pltpu.SemaphoreType.DMA,
        pltpu.VMEM_SHARED((N, D), jnp.float32),
        pltpu.VMEM_SHARED((N,), jnp.float32),
    ],
)(_edge_kernel_body)


def _norm_body(acc_ref, den_ref, out_ref):
    a = acc_ref[0] + acc_ref[1]
    d = jnp.sum(den_ref[...], axis=1)
    inv = jnp.where(d > 0, 1.0 / d, 0.0)
    out_ref[...] = a * inv[:, None]


def _normalize(acc, den):
    grid = (N // ROWS_BLK,)
    return pl.pallas_call(
        _norm_body,
        grid=grid,
        in_specs=[
            pl.BlockSpec((NC, ROWS_BLK, D), lambda i: (0, i, 0)),
            pl.BlockSpec((ROWS_BLK, NC), lambda i: (i, 0)),
        ],
        out_specs=pl.BlockSpec((ROWS_BLK, D), lambda i: (i, 0)),
        out_shape=jax.ShapeDtypeStruct((N, D), jnp.float32),
    )(acc, den)


def kernel(feat, edge_index, W_fc, attn_l, attn_r):
    fs, el8, er8 = _project(feat, W_fc, attn_l, attn_r)
    elr = jnp.stack([el8[:, 0], er8[:, 0]])
    eidx = jnp.stack(
        [edge_index[0].reshape(NW, NCH, CH),
         edge_index[1].reshape(NW, NCH, CH)], axis=2)
    acc, den = _edge_kernel(fs, elr, eidx)
    return _normalize(acc, den.T)


# speedup vs baseline: 1.2069x; 1.0110x over previous
import functools

import jax
import jax.numpy as jnp
from jax import lax
from jax.experimental import pallas as pl
from jax.experimental.pallas import tpu as pltpu
from jax.experimental.pallas import tpu_sc as plsc

N = 10000
D = 128
E = 320000
NEG_SLOPE = 0.2

NC = 2
NS = 16
NW = NC * NS
EPW = E // NW
CH = 80
NCH = EPW // CH
RPT = N // NS
ROWS_BLK = 1000


def _proj_body(feat_ref, w_ref, al_ref, ar_ref, fs_ref, elr_ref):
    x = feat_ref[...]
    w = w_ref[...]
    fs = lax.dot_general(x, w, (((1,), (1,)), ((), ())),
                         preferred_element_type=jnp.float32)
    fs_ref[...] = fs
    el = jnp.sum(fs * al_ref[...], axis=1)
    er = jnp.sum(fs * ar_ref[...], axis=1)
    elr_ref[...] = jnp.stack([el, er], axis=1)


def _project(feat, w_fc, attn_l, attn_r):
    grid = (N // ROWS_BLK,)
    return pl.pallas_call(
        _proj_body,
        grid=grid,
        in_specs=[
            pl.BlockSpec((ROWS_BLK, D), lambda i: (i, 0)),
            pl.BlockSpec((D, D), lambda i: (0, 0)),
            pl.BlockSpec((1, D), lambda i: (0, 0)),
            pl.BlockSpec((1, D), lambda i: (0, 0)),
        ],
        out_specs=[
            pl.BlockSpec((ROWS_BLK, D), lambda i: (i, 0)),
            pl.BlockSpec((ROWS_BLK, 2), lambda i: (i, 0)),
        ],
        out_shape=[
            jax.ShapeDtypeStruct((N, D), jnp.float32),
            jax.ShapeDtypeStruct((N, 2), jnp.float32),
        ],
    )(feat, w_fc, attn_l, attn_r)


def _edge_kernel_body(fs_hbm, elr_hbm, eidx_hbm,
                      acc_out, den_out,
                      elr_v, idxo, w_c, buf_a, buf_b,
                      gsem_a, gsem_b, ssem_a, ssem_b,
                      isem0, isem1, isem2, isem3, dsem_a, dsem_b,
                      acc_sh, den_sh):
    cid = lax.axis_index("c")
    sid = lax.axis_index("s")
    wid = cid * NS + sid
    bufs = (buf_a, buf_b)
    gsems = (gsem_a, gsem_b)
    ssems = (ssem_a, ssem_b)
    isems = (isem0, isem1, isem2, isem3)
    dsems = (dsem_a, dsem_b)

    pltpu.sync_copy(elr_hbm, elr_v)

    zero16 = jnp.zeros((16,), jnp.float32)
    czero = jnp.zeros((16,), jnp.int32)
    cone = czero + 1

    for q in range(2 * CH // 16):
        w_c[q // (CH // 16), pl.ds((q % (CH // 16)) * 16, 16)] = zero16

    def _zero_buf(r, c):
        for j in range(D // 16):
            buf_a[r, pl.ds(j * 16, 16)] = zero16
        return c
    lax.fori_loop(0, CH, _zero_buf, 0)

    NZ = N // CH
    for k in range((NZ + NS - 1) // NS):
        c = sid + k * NS
        @pl.when(c < NZ)
        def _z():
            off = pl.multiple_of(c * CH, 8)
            pltpu.async_copy(buf_a, acc_sh.at[pl.ds(off, CH)], gsem_a)
            pltpu.async_copy(w_c.at[0], den_sh.at[pl.ds(off, CH)], gsem_b)
    for k in range((NZ + NS - 1) // NS):
        c = sid + k * NS
        @pl.when(c < NZ)
        def _zw():
            off = pl.multiple_of(c * CH, 8)
            pltpu.make_async_copy(buf_a, acc_sh.at[pl.ds(off, CH)],
                                  gsem_a).wait()
            pltpu.make_async_copy(w_c.at[0], den_sh.at[pl.ds(off, CH)],
                                  gsem_b).wait()

    plsc.subcore_barrier()

    def _idx_sync(k, s):
        pltpu.sync_copy(eidx_hbm.at[wid, k], idxo.at[s])

    def _idx_start(k, s):
        pltpu.async_copy(eidx_hbm.at[wid, k], idxo.at[s], isems[s])

    def _idx_wait(k, s):
        pltpu.make_async_copy(eidx_hbm.at[wid, k], idxo.at[s],
                              isems[s]).wait()

    def _gather_start(s, p):
        pltpu.async_copy(fs_hbm.at[idxo.at[s, 0]], bufs[p], gsems[p])

    def _gather_wait(s, p):
        pltpu.make_async_copy(fs_hbm.at[idxo.at[s, 0]], bufs[p],
                              gsems[p]).wait()

    def _scatter_start(s, p):
        pltpu.async_copy(bufs[p], acc_sh.at[idxo.at[s, 1]], ssems[p],
                         add=True)

    def _scatter_wait(s, p):
        pltpu.make_async_copy(bufs[p], acc_sh.at[idxo.at[s, 1]],
                              ssems[p]).wait()

    def _den_start(s, p):
        pltpu.async_copy(w_c.at[p], den_sh.at[idxo.at[s, 1]], dsems[p],
                         add=True)

    def _den_wait(s, p):
        pltpu.make_async_copy(w_c.at[p], den_sh.at[idxo.at[s, 1]],
                              dsems[p]).wait()

    def _scale(s, p):
        buf = bufs[p]

        def _q(q, cc):
            sv = idxo[s, 0, pl.ds(q * 16, 16)]
            dv = idxo[s, 1, pl.ds(q * 16, 16)]
            e = (plsc.load_gather(elr_v, [sv + sv])
                 + plsc.load_gather(elr_v, [dv + dv + cone]))
            e = jnp.where(e > 0, e, NEG_SLOPE * e)
            wv = jnp.exp(e)
            w_c[p, pl.ds(q * 16, 16)] = wv
            for i in range(16):
                r = q * 16 + i
                ws = wv[i]
                for j in range(D // 16):
                    buf[r, pl.ds(j * 16, 16)] = buf[r, pl.ds(j * 16, 16)] * ws
            return cc
        lax.fori_loop(0, CH // 16, _q, 0)

    def _pipe(k, s, p, prefetch=True):
        sprev = (s + 3) & 3
        sprev2 = (s + 2) & 3
        _scatter_wait(sprev2, p)
        _den_wait(sprev2, p)
        _idx_wait(k, s)
        _gather_start(s, p)
        if prefetch:
            _idx_start(k + 2, sprev2)
        _gather_wait(sprev, 1 - p)
        _scale(sprev, 1 - p)
        _scatter_start(sprev, 1 - p)
        _den_start(sprev, 1 - p)

    _idx_sync(0, 0)
    _idx_sync(1, 1)
    _gather_start(0, 0)
    _idx_start(2, 2)
    _idx_start(3, 3)
    _gather_start(1, 1)
    _gather_wait(0, 0)
    _scale(0, 0)
    _scatter_start(0, 0)
    _den_start(0, 0)

    def _quad(t, c):
        k0 = 2 + 4 * t
        _pipe(k0, 2, 0)
        _pipe(k0 + 1, 3, 1)
        _pipe(k0 + 2, 0, 0)
        _pipe(k0 + 3, 1, 1)
        return c
    lax.fori_loop(0, (NCH - 5) // 4, _quad, 0)

    _pipe(NCH - 3, (NCH - 3) & 3, (NCH - 3) & 1)
    _pipe(NCH - 2, (NCH - 2) & 3, (NCH - 2) & 1, prefetch=False)
    _pipe(NCH - 1, (NCH - 1) & 3, (NCH - 1) & 1, prefetch=False)
    sl, pl_ = (NCH - 1) & 3, (NCH - 1) & 1
    _gather_wait(sl, pl_)
    _scale(sl, pl_)
    _scatter_start(sl, pl_)
    _den_start(sl, pl_)
    _scatter_wait((NCH - 2) & 3, (NCH - 2) & 1)
    _den_wait((NCH - 2) & 3, (NCH - 2) & 1)
    _scatter_wait(sl, pl_)
    _den_wait(sl, pl_)

    plsc.subcore_barrier()

    for k in range((NZ + NS - 1) // NS):
        c = sid + k * NS
        @pl.when(c < NZ)
        def _wb():
            off = pl.multiple_of(c * CH, 8)
            pltpu.async_copy(acc_sh.at[pl.ds(off, CH)],
                             acc_out.at[cid, pl.ds(off, CH)], gsem_a)
            pltpu.async_copy(den_sh.at[pl.ds(off, CH)],
                             den_out.at[cid, pl.ds(off, CH)], gsem_b)
    for k in range((NZ + NS - 1) // NS):
        c = sid + k * NS
        @pl.when(c < NZ)
        def _wbw():
            off = pl.multiple_of(c * CH, 8)
            pltpu.make_async_copy(acc_sh.at[pl.ds(off, CH)],
                                  acc_out.at[cid, pl.ds(off, CH)],
                                  gsem_a).wait()
            pltpu.make_async_copy(den_sh.at[pl.ds(off, CH)],
                                  den_out.at[cid, pl.ds(off, CH)],
                                  gsem_b).wait()


_edge_kernel = functools.partial(
    pl.kernel,
    out_type=(jax.ShapeDtypeStruct((NC, N, D), jnp.float32),
              jax.ShapeDtypeStruct((NC, N), jnp.float32)),
    mesh=plsc.VectorSubcoreMesh(core_axis_name="c", subcore_axis_name="s"),
    compiler_params=pltpu.CompilerParams(needs_layout_passes=False,
                                         use_tc_tiling_on_sc=False),
    scratch_types=[
        pltpu.VMEM((2 * N,), jnp.float32),
        pltpu.VMEM((4, 2, CH), jnp.int32),
        pltpu.VMEM((2, CH), jnp.float32),
        pltpu.VMEM((CH, D), jnp.float32),
        pltpu.VMEM((CH, D), jnp.float32),
        pltpu.SemaphoreType.DMA,
        pltpu.SemaphoreType.DMA,
        pltpu.SemaphoreType.DMA,
        pltpu.SemaphoreType.DMA,
        pltpu.SemaphoreType.DMA,
        pltpu.SemaphoreType.DMA,
        pltpu.SemaphoreType.DMA,
        pltpu.SemaphoreType.DMA,
        pltpu.SemaphoreType.DMA,
        pltpu.SemaphoreType.DMA,
        pltpu.VMEM_SHARED((N, D), jnp.float32),
        pltpu.VMEM_SHARED((N,), jnp.float32),
    ],
)(_edge_kernel_body)


def _norm_body(acc_ref, den_ref, out_ref):
    a = acc_ref[0] + acc_ref[1]
    d = jnp.sum(den_ref[...], axis=1)
    inv = jnp.where(d > 0, 1.0 / d, 0.0)
    out_ref[...] = a * inv[:, None]


def _normalize(acc, den):
    grid = (N // ROWS_BLK,)
    return pl.pallas_call(
        _norm_body,
        grid=grid,
        in_specs=[
            pl.BlockSpec((NC, ROWS_BLK, D), lambda i: (0, i, 0)),
            pl.BlockSpec((ROWS_BLK, NC), lambda i: (i, 0)),
        ],
        out_specs=pl.BlockSpec((ROWS_BLK, D), lambda i: (i, 0)),
        out_shape=jax.ShapeDtypeStruct((N, D), jnp.float32),
    )(acc, den)


def kernel(feat, edge_index, W_fc, attn_l, attn_r):
    fs, elr = _project(feat, W_fc, attn_l, attn_r)
    eidx = jnp.stack(
        [edge_index[0].reshape(NW, NCH, CH),
         edge_index[1].reshape(NW, NCH, CH)], axis=2)
    acc, den = _edge_kernel(fs, elr.reshape(-1), eidx)
    return _normalize(acc, den.T)
